# Initial kernel scaffold; baseline (speedup 1.0000x reference)
import functools

import jax
import jax.numpy as jnp
from jax import lax
from jax.experimental import pallas as pl
from jax.experimental.pallas import tpu as pltpu

N = 512
E = 2048
KH = 128


def _elu(v):
    return jnp.where(v > 0, v, jnp.exp(jnp.where(v > 0, 0.0, v)) - 1.0)


def _edge_h_body(ea_ref, w1a, b1a, w1b, b1b, w1c, b1c, ha, hb, hc):
    ea = ea_ref[...]
    ha[...] = jnp.maximum(jnp.dot(ea, w1a[...], preferred_element_type=jnp.float32) + b1a[...], 0.0)
    hb[...] = jnp.maximum(jnp.dot(ea, w1b[...], preferred_element_type=jnp.float32) + b1b[...], 0.0)
    hc[...] = jnp.maximum(jnp.dot(ea, w1c[...], preferred_element_type=jnp.float32) + b1c[...], 0.0)


def _edge_h(edge_attr, p1, p2, p3):
    outs = [jax.ShapeDtypeStruct((E, KH), jnp.float32)] * 3
    return pl.pallas_call(_edge_h_body, out_shape=outs)(
        edge_attr,
        p1['W1'], p1['b1'].reshape(1, KH),
        p2['W1'], p2['b1'].reshape(1, KH),
        p3['W1'], p3['b1'].reshape(1, KH),
    )


def _gather_body(x_ref, src_ref, out_ref):
    ids = lax.broadcasted_iota(jnp.int32, (E, N), 1)
    onehot = (ids == src_ref[...]).astype(jnp.float32)
    out_ref[...] = jnp.dot(onehot, x_ref[...], preferred_element_type=jnp.float32)


def _gather(x_pad, src_col):
    d = x_pad.shape[1]
    return pl.pallas_call(
        _gather_body,
        out_shape=jax.ShapeDtypeStruct((E, d), jnp.float32),
    )(x_pad, src_col)


def _msgs_body(kb, dpad, h_ref, xs_ref, w2_ref, b2_ref, out_ref):
    k = pl.program_id(0)

    @pl.when(k == 0)
    def _():
        out_ref[...] = jnp.dot(xs_ref[...], b2_ref[...], preferred_element_type=jnp.float32)

    xs = xs_ref[...]
    acc = out_ref[...]
    for kk in range(kb):
        scaled = xs * h_ref[:, kk:kk + 1]
        acc = acc + jnp.dot(scaled, w2_ref[kk * dpad:(kk + 1) * dpad, :],
                            preferred_element_type=jnp.float32)
    out_ref[...] = acc


def _msgs(h, xs, w2v, b2r, m_out, kb):
    dpad = xs.shape[1]
    kt = KH // kb
    return pl.pallas_call(
        functools.partial(_msgs_body, kb, dpad),
        grid=(kt,),
        in_specs=[
            pl.BlockSpec((E, kb), lambda k: (0, k)),
            pl.BlockSpec((E, dpad), lambda k: (0, 0)),
            pl.BlockSpec((kb * dpad, m_out), lambda k: (k, 0)),
            pl.BlockSpec((dpad, m_out), lambda k: (0, 0)),
        ],
        out_specs=pl.BlockSpec((E, m_out), lambda k: (0, 0)),
        out_shape=jax.ShapeDtypeStruct((E, m_out), jnp.float32),
    )(h, xs, w2v, b2r)


def _scatter_body(msgs_ref, dst_ref, out0_ref, out1_ref):
    half = E // 2
    ids = lax.broadcasted_iota(jnp.int32, (N, half), 0)
    oh0 = (ids == dst_ref[:, :half]).astype(jnp.float32)
    oh1 = (ids == dst_ref[:, half:]).astype(jnp.float32)
    out0_ref[...] = jnp.dot(oh0, msgs_ref[:half, :], preferred_element_type=jnp.float32)
    out1_ref[...] = jnp.dot(oh1, msgs_ref[half:, :], preferred_element_type=jnp.float32)


def _scatter(msgs, dst_row, m_out):
    outs = [jax.ShapeDtypeStruct((N, m_out), jnp.float32)] * 2
    return pl.pallas_call(_scatter_body, out_shape=outs)(msgs, dst_row)


def _combine_body(a0, a1, x_ref, root_ref, bias_ref, out_ref):
    pre = (a0[...] + a1[...]
           + jnp.dot(x_ref[...], root_ref[...], preferred_element_type=jnp.float32)
           + bias_ref[...])
    out_ref[...] = _elu(pre)


def _combine(agg0, agg1, x_pad, root_pad, bias, m_out):
    return pl.pallas_call(
        _combine_body,
        out_shape=jax.ShapeDtypeStruct((N, m_out), jnp.float32),
    )(agg0, agg1, x_pad, root_pad, bias.reshape(1, m_out))


def _head_body(a0, a1, x_ref, root_ref, bias_ref,
               w1_ref, b1_ref, w2_ref, b2_ref, w3_ref, b3_ref, out_ref):
    h = _elu(a0[...] + a1[...]
             + jnp.dot(x_ref[...], root_ref[...], preferred_element_type=jnp.float32)
             + bias_ref[...])
    h = _elu(jnp.dot(h, w1_ref[...], preferred_element_type=jnp.float32) + b1_ref[...])
    h = _elu(jnp.dot(h, w2_ref[...], preferred_element_type=jnp.float32) + b2_ref[...])
    out_ref[...] = jnp.sum(h * w3_ref[...], axis=1, keepdims=True) + b3_ref[...]


def _head(agg0, agg1, x_pad, root_pad, bias, params):
    return pl.pallas_call(
        _head_body,
        out_shape=jax.ShapeDtypeStruct((N, 1), jnp.float32),
    )(agg0, agg1, x_pad, root_pad, bias.reshape(1, 512),
      params['fc1_W'], params['fc1_b'].reshape(1, 256),
      params['fc2_W'], params['fc2_b'].reshape(1, 128),
      params['fc3_W'].reshape(1, 128), params['fc3_b'].reshape(1, 1))


def _pad_cols(a, dpad):
    d = a.shape[-1]
    if d == dpad:
        return a
    return jnp.pad(a, [(0, 0)] * (a.ndim - 1) + [(0, dpad - d)])


def _prep_conv(p, m_in, m_out, dpad):
    w2r = p['W2'].reshape(KH, m_in, m_out)
    w2r = jnp.pad(w2r, ((0, 0), (0, dpad - m_in), (0, 0))) if dpad != m_in else w2r
    w2v = w2r.reshape(KH * dpad, m_out)
    b2r = p['b2'].reshape(m_in, m_out)
    b2r = jnp.pad(b2r, ((0, dpad - m_in), (0, 0))) if dpad != m_in else b2r
    root = p['root']
    root = jnp.pad(root, ((0, dpad - m_in), (0, 0))) if dpad != m_in else root
    return w2v, b2r, root


def kernel(x, edge_index, edge_attr, params):
    src_col = edge_index[0].reshape(E, 1)
    dst_row = edge_index[1].reshape(1, E)

    h1, h2, h3 = _edge_h(edge_attr, params['conv1'], params['conv2'], params['conv3'])

    layers = [
        ('conv1', 37, 128, 48, 16, h1),
        ('conv2', 128, 256, 128, 8, h2),
        ('conv3', 256, 512, 256, 8, h3),
    ]

    cur = _pad_cols(x, 48)
    out = None
    for name, m_in, m_out, dpad, kb, h in layers:
        p = params[name]
        w2v, b2r, root = _prep_conv(p, m_in, m_out, dpad)
        xs = _gather(cur, src_col)
        msgs = _msgs(h, xs, w2v, b2r, m_out, kb)
        agg0, agg1 = _scatter(msgs, dst_row, m_out)
        if name == 'conv3':
            out = _head(agg0, agg1, cur, root, p['bias'], params)
        else:
            cur = _combine(agg0, agg1, cur, root, p['bias'], m_out)

    return out.reshape(-1)


# speedup vs baseline: 3.3431x; 3.3431x over previous
import functools

import jax
import jax.numpy as jnp
from jax import lax
from jax.experimental import pallas as pl
from jax.experimental.pallas import tpu as pltpu

N = 512
E = 2048
KH = 128


def _elu(v):
    return jnp.where(v > 0, v, jnp.exp(jnp.where(v > 0, 0.0, v)) - 1.0)


def _edge_h_body(ea_ref, w1a, b1a, w1b, b1b, w1c, b1c, ha, hb, hc):
    ea = ea_ref[...]
    ha[...] = jnp.maximum(jnp.dot(ea, w1a[...], preferred_element_type=jnp.float32) + b1a[...], 0.0)
    hb[...] = jnp.maximum(jnp.dot(ea, w1b[...], preferred_element_type=jnp.float32) + b1b[...], 0.0)
    hc[...] = jnp.maximum(jnp.dot(ea, w1c[...], preferred_element_type=jnp.float32) + b1c[...], 0.0)


def _edge_h(edge_attr, p1, p2, p3):
    outs = [jax.ShapeDtypeStruct((E, KH), jnp.float32)] * 3
    return pl.pallas_call(_edge_h_body, out_shape=outs)(
        edge_attr,
        p1['W1'], p1['b1'].reshape(1, KH),
        p2['W1'], p2['b1'].reshape(1, KH),
        p3['W1'], p3['b1'].reshape(1, KH),
    )


def _gather_body(x_ref, src_ref, out_ref):
    ids = lax.broadcasted_iota(jnp.int32, (E, N), 1)
    onehot = (ids == src_ref[...]).astype(jnp.float32)
    out_ref[...] = jnp.dot(onehot, x_ref[...], preferred_element_type=jnp.float32)


def _gather(x_pad, src_col):
    d = x_pad.shape[1]
    return pl.pallas_call(
        _gather_body,
        out_shape=jax.ShapeDtypeStruct((E, d), jnp.float32),
    )(x_pad, src_col)


def _msgs_body(kb, dpad, h_ref, xs_ref, w2_ref, b2_ref, out_ref):
    k = pl.program_id(0)

    @pl.when(k == 0)
    def _():
        out_ref[...] = jnp.dot(xs_ref[...], b2_ref[...], preferred_element_type=jnp.float32)

    xs = xs_ref[...]
    hb = h_ref[0]
    acc = out_ref[...]
    for kk in range(kb):
        scaled = xs * hb[:, kk:kk + 1]
        acc = acc + jnp.dot(scaled, w2_ref[kk * dpad:(kk + 1) * dpad, :],
                            preferred_element_type=jnp.float32)
    out_ref[...] = acc


def _msgs(h, xs, w2v, b2r, m_out, kb):
    dpad = xs.shape[1]
    kt = KH // kb
    h3 = h.reshape(E, kt, kb).transpose(1, 0, 2)
    return pl.pallas_call(
        functools.partial(_msgs_body, kb, dpad),
        grid=(kt,),
        in_specs=[
            pl.BlockSpec((1, E, kb), lambda k: (k, 0, 0)),
            pl.BlockSpec((E, dpad), lambda k: (0, 0)),
            pl.BlockSpec((kb * dpad, m_out), lambda k: (k, 0)),
            pl.BlockSpec((dpad, m_out), lambda k: (0, 0)),
        ],
        out_specs=pl.BlockSpec((E, m_out), lambda k: (0, 0)),
        out_shape=jax.ShapeDtypeStruct((E, m_out), jnp.float32),
    )(h3, xs, w2v, b2r)


def _scatter_body(msgs_ref, dst_ref, out0_ref, out1_ref):
    half = E // 2
    ids = lax.broadcasted_iota(jnp.int32, (N, half), 0)
    oh0 = (ids == dst_ref[:, :half]).astype(jnp.float32)
    oh1 = (ids == dst_ref[:, half:]).astype(jnp.float32)
    out0_ref[...] = jnp.dot(oh0, msgs_ref[:half, :], preferred_element_type=jnp.float32)
    out1_ref[...] = jnp.dot(oh1, msgs_ref[half:, :], preferred_element_type=jnp.float32)


def _scatter(msgs, dst_row, m_out):
    outs = [jax.ShapeDtypeStruct((N, m_out), jnp.float32)] * 2
    return pl.pallas_call(_scatter_body, out_shape=outs)(msgs, dst_row)


def _combine_body(a0, a1, x_ref, root_ref, bias_ref, out_ref):
    pre = (a0[...] + a1[...]
           + jnp.dot(x_ref[...], root_ref[...], preferred_element_type=jnp.float32)
           + bias_ref[...])
    out_ref[...] = _elu(pre)


def _combine(agg0, agg1, x_pad, root_pad, bias, m_out):
    return pl.pallas_call(
        _combine_body,
        out_shape=jax.ShapeDtypeStruct((N, m_out), jnp.float32),
    )(agg0, agg1, x_pad, root_pad, bias.reshape(1, m_out))


def _head_body(a0, a1, x_ref, root_ref, bias_ref,
               w1_ref, b1_ref, w2_ref, b2_ref, w3_ref, b3_ref, out_ref):
    h = _elu(a0[...] + a1[...]
             + jnp.dot(x_ref[...], root_ref[...], preferred_element_type=jnp.float32)
             + bias_ref[...])
    h = _elu(jnp.dot(h, w1_ref[...], preferred_element_type=jnp.float32) + b1_ref[...])
    h = _elu(jnp.dot(h, w2_ref[...], preferred_element_type=jnp.float32) + b2_ref[...])
    out_ref[...] = jnp.sum(h * w3_ref[...], axis=1, keepdims=True) + b3_ref[...]


def _head(agg0, agg1, x_pad, root_pad, bias, params):
    return pl.pallas_call(
        _head_body,
        out_shape=jax.ShapeDtypeStruct((N, 1), jnp.float32),
    )(agg0, agg1, x_pad, root_pad, bias.reshape(1, 512),
      params['fc1_W'], params['fc1_b'].reshape(1, 256),
      params['fc2_W'], params['fc2_b'].reshape(1, 128),
      params['fc3_W'].reshape(1, 128), params['fc3_b'].reshape(1, 1))


def _pad_cols(a, dpad):
    d = a.shape[-1]
    if d == dpad:
        return a
    return jnp.pad(a, [(0, 0)] * (a.ndim - 1) + [(0, dpad - d)])


def _prep_conv(p, m_in, m_out, dpad):
    w2r = p['W2'].reshape(KH, m_in, m_out)
    w2r = jnp.pad(w2r, ((0, 0), (0, dpad - m_in), (0, 0))) if dpad != m_in else w2r
    w2v = w2r.reshape(KH * dpad, m_out)
    b2r = p['b2'].reshape(m_in, m_out)
    b2r = jnp.pad(b2r, ((0, dpad - m_in), (0, 0))) if dpad != m_in else b2r
    root = p['root']
    root = jnp.pad(root, ((0, dpad - m_in), (0, 0))) if dpad != m_in else root
    return w2v, b2r, root


def kernel(x, edge_index, edge_attr, params):
    src_col = edge_index[0].reshape(E, 1)
    dst_row = edge_index[1].reshape(1, E)

    h1, h2, h3 = _edge_h(edge_attr, params['conv1'], params['conv2'], params['conv3'])

    layers = [
        ('conv1', 37, 128, 48, 16, h1),
        ('conv2', 128, 256, 128, 8, h2),
        ('conv3', 256, 512, 256, 8, h3),
    ]

    cur = _pad_cols(x, 48)
    out = None
    for name, m_in, m_out, dpad, kb, h in layers:
        p = params[name]
        w2v, b2r, root = _prep_conv(p, m_in, m_out, dpad)
        xs = _gather(cur, src_col)
        msgs = _msgs(h, xs, w2v, b2r, m_out, kb)
        agg0, agg1 = _scatter(msgs, dst_row, m_out)
        if name == 'conv3':
            out = _head(agg0, agg1, cur, root, p['bias'], params)
        else:
            cur = _combine(agg0, agg1, cur, root, p['bias'], m_out)

    return out.reshape(-1)
